# Initial kernel scaffold; baseline (speedup 1.0000x reference)
import jax
import jax.numpy as jnp
from jax.experimental import pallas as pl


def kernel(mask, edge, iter):
    raise NotImplementedError("write your pallas kernel here")


# speedup vs baseline: 19.3006x; 19.3006x over previous
import jax
import jax.numpy as jnp
from jax.experimental import pallas as pl
from jax.experimental.pallas import tpu as pltpu

_THETA = 40.0
_KS = (1, 2, 4, 8, 16, 32, 64)


def _shift_down(a, k, axis):
    n = a.shape[axis]
    zeros = jnp.zeros_like(jax.lax.slice_in_dim(a, 0, k, axis=axis))
    return jnp.concatenate(
        [zeros, jax.lax.slice_in_dim(a, 0, n - k, axis=axis)], axis=axis)


def _shift_up(a, k, axis):
    n = a.shape[axis]
    zeros = jnp.zeros_like(jax.lax.slice_in_dim(a, 0, k, axis=axis))
    return jnp.concatenate(
        [jax.lax.slice_in_dim(a, k, n, axis=axis), zeros], axis=axis)


def _build_ladder(d0, shift, axis):
    ladder = []
    dcur = d0
    for k in _KS:
        ladder.append(dcur)
        if k != _KS[-1]:
            dcur = dcur * shift(dcur, k, axis)
    return ladder


def _scan(x, ladder, shift, axis):
    f = x
    for dcur, k in zip(ladder, _KS):
        f = f + dcur * shift(f, k, axis)
    return f


def _ccedge_body(it_ref, mask_ref, edge_ref, out_ref):
    x0 = mask_ref[...]
    e = jnp.maximum(edge_ref[...], 0.0)
    d = jnp.exp(-_THETA * e)

    iota_h = jax.lax.broadcasted_iota(jnp.int32, d.shape, 2)
    iota_w = jax.lax.broadcasted_iota(jnp.int32, d.shape, 3)
    d0_fh = jnp.where(iota_h == 0, 0.0, d)
    d0_fw = jnp.where(iota_w == 0, 0.0, d)
    d0_bh = _shift_up(d, 1, 2)
    d0_bw = _shift_up(d, 1, 3)

    lad_fh = _build_ladder(d0_fh, _shift_down, 2)
    lad_bh = _build_ladder(d0_bh, _shift_up, 2)
    lad_fw = _build_ladder(d0_fw, _shift_down, 3)
    lad_bw = _build_ladder(d0_bw, _shift_up, 3)

    def crisscross(x):
        fh = _scan(x, lad_fh, _shift_down, 2)
        bh = _scan(x, lad_bh, _shift_up, 2)
        fw = _scan(x, lad_fw, _shift_down, 3)
        bw = _scan(x, lad_bw, _shift_up, 3)
        return fh + bh + fw + bw - 3.0 * x

    ones = jnp.ones_like(d)
    rz = 1.0 / crisscross(ones)

    def one_iter(_, x):
        return crisscross(x) * rz

    out_ref[...] = jax.lax.fori_loop(0, it_ref[0], one_iter, x0)


def kernel(mask, edge, iter):
    it = jnp.asarray(iter, jnp.int32).reshape(1)
    return pl.pallas_call(
        _ccedge_body,
        out_shape=jax.ShapeDtypeStruct(mask.shape, mask.dtype),
        in_specs=[
            pl.BlockSpec(memory_space=pltpu.SMEM),
            pl.BlockSpec(memory_space=pltpu.VMEM),
            pl.BlockSpec(memory_space=pltpu.VMEM),
        ],
        out_specs=pl.BlockSpec(memory_space=pltpu.VMEM),
    )(it, mask, edge)
